# baseline (speedup 1.0000x reference)
import jax
import jax.numpy as jnp
from jax import lax
from jax.experimental import pallas as pl
from jax.experimental.pallas import tpu as pltpu
from jax.experimental.pallas import tpu_sc as plsc

EMBED_DIM = 128
BATCH = 16384
VOCAB_ROWS = 1489
VOCAB_PAD = 1536
NUM_CORES = 2
NUM_SUBCORES = 16
NUM_WORKERS = NUM_CORES * NUM_SUBCORES
BPW = BATCH // NUM_WORKERS
CHUNK = 128
NCHUNK = BPW // CHUNK
NBUF = 4
TROWS = 96

TC_BLOCK = 1024


def _sc_gather_body(table_hbm, ov_hbm, out_ov,
                    table_sh, idx_all, rows0, rows1, rows2, rows3,
                    isem, tsem, gsem0, gsem1, gsem2, gsem3,
                    wsem0, wsem1, wsem2, wsem3):
    sid = lax.axis_index("s")
    wid = sid * NUM_CORES + lax.axis_index("c")
    base = wid * BPW

    row_bufs = (rows0, rows1, rows2, rows3)
    gsems = (gsem0, gsem1, gsem2, gsem3)
    wsems = (wsem0, wsem1, wsem2, wsem3)

    tail = sid == NUM_SUBCORES - 1

    @pl.when(jnp.logical_not(tail))
    def _stage_main():
        pltpu.async_copy(
            table_hbm.at[pl.ds(sid * TROWS, TROWS)],
            table_sh.at[pl.ds(sid * TROWS, TROWS)], tsem).wait()

    @pl.when(tail)
    def _stage_tail():
        pltpu.async_copy(
            table_hbm.at[pl.ds(15 * TROWS, VOCAB_ROWS - 15 * TROWS)],
            table_sh.at[pl.ds(15 * TROWS, VOCAB_ROWS - 15 * TROWS)],
            tsem).wait()

    pltpu.async_copy(
        ov_hbm.at[pl.ds(wid * NCHUNK, NCHUNK)], idx_all, isem).wait()
    plsc.subcore_barrier()

    gathers = [None] * NBUF
    writes = [None] * NBUF
    for k in range(NCHUNK):
        slot = k % NBUF
        if writes[slot] is not None:
            writes[slot].wait()
        gathers[slot] = pltpu.async_copy(
            table_sh.at[idx_all.at[k]], row_bufs[slot], gsems[slot])
        if k >= NBUF - 1:
            pk = k - (NBUF - 1)
            pslot = pk % NBUF
            gathers[pslot].wait()
            writes[pslot] = pltpu.async_copy(
                row_bufs[pslot],
                out_ov.at[pl.ds(base + pk * CHUNK, CHUNK)], wsems[pslot])
    for pk in range(max(0, NCHUNK - (NBUF - 1)), NCHUNK):
        pslot = pk % NBUF
        gathers[pslot].wait()
        writes[pslot] = pltpu.async_copy(
            row_bufs[pslot],
            out_ov.at[pl.ds(base + pk * CHUNK, CHUNK)], wsems[pslot])
    for w in writes:
        if w is not None:
            w.wait()


def _tc_onehot_body(idx_ref, hilo_ref, out_ref):
    idxv = idx_ref[...]
    iota = lax.broadcasted_iota(jnp.int32, (TC_BLOCK, VOCAB_PAD), 1)
    oh = (iota == idxv).astype(jnp.bfloat16)
    acc = jnp.dot(oh, hilo_ref[...], preferred_element_type=jnp.float32)
    out_ref[...] = acc[:, :EMBED_DIM] + acc[:, EMBED_DIM:]


@jax.jit
def kernel(overlap, scene, embed_table):
    ov = overlap.astype(jnp.int32).reshape(BATCH // CHUNK, CHUNK)
    sc = scene.astype(jnp.int32).reshape(BATCH, 1)

    hi = embed_table.astype(jnp.bfloat16)
    lo = (embed_table - hi.astype(jnp.float32)).astype(jnp.bfloat16)
    hilo = jnp.concatenate([hi, lo], axis=1)
    hilo = jnp.pad(hilo, ((0, VOCAB_PAD - VOCAB_ROWS), (0, 0)))

    out_sds = jax.ShapeDtypeStruct((BATCH, EMBED_DIM), jnp.float32)

    sc_run = pl.kernel(
        _sc_gather_body,
        out_type=out_sds,
        mesh=plsc.VectorSubcoreMesh(core_axis_name="c", subcore_axis_name="s"),
        scratch_types=(
            [pltpu.VMEM_SHARED((VOCAB_ROWS, EMBED_DIM), jnp.float32)]
            + [pltpu.VMEM((NCHUNK, CHUNK), jnp.int32)]
            + [pltpu.VMEM((CHUNK, EMBED_DIM), jnp.float32)] * NBUF
            + [pltpu.SemaphoreType.DMA] * (2 * NBUF + 2)
        ),
    )
    overlap_emb = sc_run(embed_table, ov)

    scene_emb = pl.pallas_call(
        _tc_onehot_body,
        grid=(BATCH // TC_BLOCK,),
        in_specs=[
            pl.BlockSpec((TC_BLOCK, 1), lambda i: (i, 0)),
            pl.BlockSpec((VOCAB_PAD, 2 * EMBED_DIM), lambda i: (0, 0)),
        ],
        out_specs=pl.BlockSpec((TC_BLOCK, EMBED_DIM), lambda i: (i, 0)),
        out_shape=out_sds,
    )(sc, hilo)

    return (overlap_emb, scene_emb)


# speedup vs baseline: 1.4671x; 1.4671x over previous
import jax
import jax.numpy as jnp
from jax import lax
from jax.experimental import pallas as pl
from jax.experimental.pallas import tpu as pltpu
from jax.experimental.pallas import tpu_sc as plsc

EMBED_DIM = 128
BATCH = 16384
VOCAB_ROWS = 1489
NUM_CORES = 2
NUM_SUBCORES = 16
NUM_WORKERS = NUM_CORES * NUM_SUBCORES
BPW = BATCH // NUM_WORKERS
CHUNK = 128
NCHUNK = BPW // CHUNK
TOTAL = 2 * NCHUNK
SUPER = 2 * CHUNK
NSUPER = TOTAL // 2
NBUF = 3


TROWS = 96


def _gather_body(table_hbm, ov_hbm, sc_hbm, out_ov, out_sc,
                 table_sh, idx_all, rows0, rows1, rows2,
                 isem, tsem, gsem0, gsem1, gsem2,
                 wsem0, wsem1, wsem2):
    sid = lax.axis_index("s")
    wid = sid * NUM_CORES + lax.axis_index("c")
    row0 = wid * NCHUNK
    base = wid * BPW

    row_bufs = (rows0, rows1, rows2)
    gsems = (gsem0, gsem1, gsem2)
    wsems = (wsem0, wsem1, wsem2)

    tail = sid == NUM_SUBCORES - 1

    @pl.when(jnp.logical_not(tail))
    def _stage_main():
        pltpu.async_copy(
            table_hbm.at[pl.ds(sid * TROWS, TROWS)],
            table_sh.at[pl.ds(sid * TROWS, TROWS)], tsem).wait()

    @pl.when(tail)
    def _stage_tail():
        pltpu.async_copy(
            table_hbm.at[pl.ds(15 * TROWS, VOCAB_ROWS - 15 * TROWS)],
            table_sh.at[pl.ds(15 * TROWS, VOCAB_ROWS - 15 * TROWS)],
            tsem).wait()

    cp_i0 = pltpu.async_copy(
        ov_hbm.at[pl.ds(row0, NCHUNK)], idx_all.at[pl.ds(0, NCHUNK)], isem)
    cp_i1 = pltpu.async_copy(
        sc_hbm.at[pl.ds(row0, NCHUNK)], idx_all.at[pl.ds(NCHUNK, NCHUNK)],
        isem)
    cp_i0.wait()
    cp_i1.wait()
    plsc.subcore_barrier()

    def out_ref(j):
        if j < NSUPER // 2:
            return out_ov.at[pl.ds(base + j * SUPER, SUPER)]
        return out_sc.at[pl.ds(base + (j - NSUPER // 2) * SUPER, SUPER)]

    gathers = [None] * NBUF
    writes = [None] * NBUF
    for j in range(NSUPER):
        slot = j % NBUF
        if writes[slot] is not None:
            writes[slot].wait()
        gathers[slot] = (
            pltpu.async_copy(table_sh.at[idx_all.at[2 * j]],
                             row_bufs[slot].at[pl.ds(0, CHUNK)], gsems[slot]),
            pltpu.async_copy(table_sh.at[idx_all.at[2 * j + 1]],
                             row_bufs[slot].at[pl.ds(CHUNK, CHUNK)],
                             gsems[slot]),
        )
        if j >= NBUF - 1:
            pj = j - (NBUF - 1)
            pslot = pj % NBUF
            gathers[pslot][0].wait()
            gathers[pslot][1].wait()
            writes[pslot] = pltpu.async_copy(
                row_bufs[pslot], out_ref(pj), wsems[pslot])
    for pj in range(max(0, NSUPER - (NBUF - 1)), NSUPER):
        pslot = pj % NBUF
        gathers[pslot][0].wait()
        gathers[pslot][1].wait()
        writes[pslot] = pltpu.async_copy(
            row_bufs[pslot], out_ref(pj), wsems[pslot])
    for w in writes:
        if w is not None:
            w.wait()


@jax.jit
def kernel(overlap, scene, embed_table):
    ov = overlap.astype(jnp.int32).reshape(BATCH // CHUNK, CHUNK)
    sc = scene.astype(jnp.int32).reshape(BATCH // CHUNK, CHUNK)
    out_sds = jax.ShapeDtypeStruct((BATCH, EMBED_DIM), jnp.float32)
    run = pl.kernel(
        _gather_body,
        out_type=(out_sds, out_sds),
        mesh=plsc.VectorSubcoreMesh(core_axis_name="c", subcore_axis_name="s"),
        scratch_types=(
            [pltpu.VMEM_SHARED((VOCAB_ROWS, EMBED_DIM), jnp.float32)]
            + [pltpu.VMEM((TOTAL, CHUNK), jnp.int32)]
            + [pltpu.VMEM((SUPER, EMBED_DIM), jnp.float32)] * NBUF
            + [pltpu.SemaphoreType.DMA] * (2 * NBUF + 2)
        ),
    )
    return run(embed_table, ov, sc)
